# baseline (speedup 1.0000x reference)
import functools

import jax
import jax.numpy as jnp
from jax import lax
from jax.experimental import pallas as pl
from jax.experimental.pallas import tpu as pltpu
from jax.experimental.pallas import tpu_sc as plsc

_D = 64
_T = 10
_N = 100000

_NW = 32
_W = 1536
_NSC = _NW * _W

_NBLK = 6144
_KTC = 9
_NPAD = _NSC + _KTC * _NBLK
_LANES = 16


def _sc_body(rank_hbm, hb_hbm, out_hbm, buf_v, hb_v, acc_v):
    wid = lax.axis_index("s") * 2 + lax.axis_index("c")
    col0 = wid * _W
    pltpu.sync_copy(hb_hbm, hb_v)
    pltpu.sync_copy(rank_hbm.at[:, pl.ds(col0, _W)], buf_v)

    def init_g(g, carry):
        sl = pl.ds(g * _LANES, _LANES)
        acc_v[sl] = hb_v[0, :] * buf_v[0, sl]
        return carry

    lax.fori_loop(0, _W // _LANES, init_g, 0, unroll=4)

    def mac_g(g, carry):
        sl = pl.ds(g * _LANES, _LANES)
        for k in range(1, _D):
            plsc.addupdate(acc_v.at[sl], hb_v[k, :] * buf_v[k, sl])
        return carry

    lax.fori_loop(0, _W // _LANES, mac_g, 0)
    pltpu.sync_copy(acc_v, out_hbm.at[pl.ds(col0, _W)])


@functools.partial(
    pl.kernel,
    out_type=jax.ShapeDtypeStruct((_NSC,), jnp.float32),
    mesh=plsc.VectorSubcoreMesh(core_axis_name="c", subcore_axis_name="s",
                                num_cores=2, num_subcores=16),
    scratch_types=[
        pltpu.VMEM((_D, _W), jnp.float32),
        pltpu.VMEM((_D, _LANES), jnp.float32),
        pltpu.VMEM((_W,), jnp.float32),
    ],
)
def _sc_logits(rank_hbm, hb_hbm, out_hbm, buf_v, hb_v, acc_v):
    _sc_body(rank_hbm, hb_hbm, out_hbm, buf_v, hb_v, acc_v)


def _tc_matvec(h_ref, cls_ref, rank_ref, pred_ref, logits_ref):
    i = pl.program_id(0)
    h = h_ref[:, :]
    logits_ref[:, :] = jnp.dot(h, rank_ref[:, :],
                               preferred_element_type=jnp.float32)

    @pl.when(i == _KTC - 1)
    def _pred():
        pred_ref[:, :] = jnp.dot(h, cls_ref[:, :],
                                 preferred_element_type=jnp.float32)


def _tc_softmax(sc_ref, tc_ref, attn_ref):
    cols_sc = jax.lax.broadcasted_iota(jnp.int32, (1, _NSC), 1)
    cols_tc = jax.lax.broadcasted_iota(jnp.int32, (1, _NPAD - _NSC), 1) + _NSC
    lsc = sc_ref[:, :]
    ltc = jnp.where(cols_tc < _N, tc_ref[:, :], -jnp.inf)
    m = jnp.maximum(jnp.max(lsc), jnp.max(ltc))
    esc = jnp.exp(lsc - m)
    etc = jnp.exp(ltc - m)
    s = jnp.sum(esc) + jnp.sum(etc)
    inv = 1.0 / s
    attn_ref[:, : _NSC] = esc * inv
    attn_ref[:, _NSC:] = (etc * inv)[:, : _N - _NSC]


@jax.jit
def kernel(hidden_state, theta_classification, theta_rank):
    h = hidden_state.reshape(1, _D)
    hb = jnp.broadcast_to(hidden_state.reshape(_D, 1), (_D, _LANES))
    hb = jnp.asarray(hb, dtype=jnp.float32)

    logits_sc = _sc_logits(theta_rank, hb)

    pred, logits_tc = pl.pallas_call(
        _tc_matvec,
        grid=(_KTC,),
        in_specs=[
            pl.BlockSpec((1, _D), lambda i: (0, 0)),
            pl.BlockSpec((_D, _T), lambda i: (0, 0)),
            pl.BlockSpec((_D, _NBLK), lambda i: (0, i + _NSC // _NBLK)),
        ],
        out_specs=[
            pl.BlockSpec((1, _T), lambda i: (0, 0)),
            pl.BlockSpec((1, _NBLK), lambda i: (0, i)),
        ],
        out_shape=[
            jax.ShapeDtypeStruct((1, _T), jnp.float32),
            jax.ShapeDtypeStruct((1, _NPAD - _NSC), jnp.float32),
        ],
    )(h, theta_classification, theta_rank)

    attn = pl.pallas_call(
        _tc_softmax,
        out_shape=jax.ShapeDtypeStruct((1, _N), jnp.float32),
    )(logits_sc.reshape(1, _NSC), logits_tc)

    return (pred, attn)


# speedup vs baseline: 1.4281x; 1.4281x over previous
import functools

import jax
import jax.numpy as jnp
from jax import lax
from jax.experimental import pallas as pl
from jax.experimental.pallas import tpu as pltpu
from jax.experimental.pallas import tpu_sc as plsc

_D = 64
_T = 10
_N = 100000

_NW = 32
_W = 1536
_NSC = _NW * _W

_NBLK = 6144
_KTC = 9
_NPAD = _NSC + _KTC * _NBLK
_LANES = 16


def _sc_body(rank_hbm, hb_hbm, out_hbm, buf_v, hb_v, acc_v):
    wid = lax.axis_index("s") * 2 + lax.axis_index("c")
    col0 = wid * _W
    pltpu.sync_copy(hb_hbm, hb_v)
    pltpu.sync_copy(rank_hbm.at[:, pl.ds(col0, _W)], buf_v)

    def mac_g(g, carry):
        sl = pl.ds(g * _LANES, _LANES)
        acc = [hb_v[k, :] * buf_v[k, sl] for k in range(4)]
        for k in range(4, _D):
            acc[k % 4] = acc[k % 4] + hb_v[k, :] * buf_v[k, sl]
        acc_v[sl] = (acc[0] + acc[1]) + (acc[2] + acc[3])
        return carry

    lax.fori_loop(0, _W // _LANES, mac_g, 0)
    pltpu.sync_copy(acc_v, out_hbm.at[pl.ds(col0, _W)])


@functools.partial(
    pl.kernel,
    out_type=jax.ShapeDtypeStruct((_NSC,), jnp.float32),
    mesh=plsc.VectorSubcoreMesh(core_axis_name="c", subcore_axis_name="s",
                                num_cores=2, num_subcores=16),
    scratch_types=[
        pltpu.VMEM((_D, _W), jnp.float32),
        pltpu.VMEM((_D, _LANES), jnp.float32),
        pltpu.VMEM((_W,), jnp.float32),
    ],
)
def _sc_logits(rank_hbm, hb_hbm, out_hbm, buf_v, hb_v, acc_v):
    _sc_body(rank_hbm, hb_hbm, out_hbm, buf_v, hb_v, acc_v)


def _tc_matvec(h_ref, cls_ref, rank_ref, pred_ref, logits_ref):
    i = pl.program_id(0)
    h = h_ref[:, :]
    logits_ref[:, :] = jnp.dot(h, rank_ref[:, :],
                               preferred_element_type=jnp.float32)

    @pl.when(i == _KTC - 1)
    def _pred():
        pred_ref[:, :] = jnp.dot(h, cls_ref[:, :],
                                 preferred_element_type=jnp.float32)


def _tc_softmax(sc_ref, tc_ref, attn_ref):
    cols_sc = jax.lax.broadcasted_iota(jnp.int32, (1, _NSC), 1)
    cols_tc = jax.lax.broadcasted_iota(jnp.int32, (1, _NPAD - _NSC), 1) + _NSC
    lsc = sc_ref[:, :]
    ltc = jnp.where(cols_tc < _N, tc_ref[:, :], -jnp.inf)
    m = jnp.maximum(jnp.max(lsc), jnp.max(ltc))
    esc = jnp.exp(lsc - m)
    etc = jnp.exp(ltc - m)
    s = jnp.sum(esc) + jnp.sum(etc)
    inv = 1.0 / s
    attn_ref[:, : _NSC] = esc * inv
    attn_ref[:, _NSC:] = (etc * inv)[:, : _N - _NSC]


@jax.jit
def kernel(hidden_state, theta_classification, theta_rank):
    h = hidden_state.reshape(1, _D)
    hb = jnp.broadcast_to(hidden_state.reshape(_D, 1), (_D, _LANES))
    hb = jnp.asarray(hb, dtype=jnp.float32)

    logits_sc = _sc_logits(theta_rank, hb)

    pred, logits_tc = pl.pallas_call(
        _tc_matvec,
        grid=(_KTC,),
        in_specs=[
            pl.BlockSpec((1, _D), lambda i: (0, 0)),
            pl.BlockSpec((_D, _T), lambda i: (0, 0)),
            pl.BlockSpec((_D, _NBLK), lambda i: (0, i + _NSC // _NBLK)),
        ],
        out_specs=[
            pl.BlockSpec((1, _T), lambda i: (0, 0)),
            pl.BlockSpec((1, _NBLK), lambda i: (0, i)),
        ],
        out_shape=[
            jax.ShapeDtypeStruct((1, _T), jnp.float32),
            jax.ShapeDtypeStruct((1, _NPAD - _NSC), jnp.float32),
        ],
    )(h, theta_classification, theta_rank)

    attn = pl.pallas_call(
        _tc_softmax,
        out_shape=jax.ShapeDtypeStruct((1, _N), jnp.float32),
    )(logits_sc.reshape(1, _NSC), logits_tc)

    return (pred, attn)


# speedup vs baseline: 5.7620x; 4.0348x over previous
import functools

import jax
import jax.numpy as jnp
from jax.experimental import pallas as pl
from jax.experimental.pallas import tpu as pltpu

_D = 64
_T = 10
_N = 100000

_FBLK = 12544
_FROWS = _FBLK // 128
_NBLKS = 8
_NSTREAMS = 2
_KSTEPS = _NBLKS // _NSTREAMS


def _store_block(e_ref, m_ref, s_ref, l, jblk, need_mask):
    l2 = l.reshape(_FROWS, 128)
    if need_mask:
        rows = jax.lax.broadcasted_iota(jnp.int32, (_FROWS, 128), 0)
        lanes = jax.lax.broadcasted_iota(jnp.int32, (_FROWS, 128), 1)
        gcol = rows * 128 + lanes + jblk * _FBLK
        l2 = jnp.where(gcol < _N, l2, -jnp.inf)
    m = jnp.max(l2)
    e2 = jnp.exp(l2 - m)
    e_ref[pl.ds(jblk * _FROWS, _FROWS), :] = e2
    m_ref[:, pl.ds(jblk * 128, 128)] = jnp.full((1, 128), m, jnp.float32)
    s_ref[:, pl.ds(jblk * 128, 128)] = jnp.full((1, 128), jnp.sum(e2),
                                                jnp.float32)


def _tc_fused(h_ref, cls_ref, r0_ref, r1_ref,
              pred_ref, attn_ref, e_ref, m_ref, s_ref):
    i = pl.program_id(0)

    @pl.when(i < _KSTEPS)
    def _phase1():
        h = h_ref[:, :]
        for s, rref in enumerate((r0_ref, r1_ref)):
            l = jnp.dot(h, rref[:, :], preferred_element_type=jnp.float32)
            jblk = s * _KSTEPS + i
            _store_block(e_ref, m_ref, s_ref, l, jblk, s == _NSTREAMS - 1)

    @pl.when(i == 0)
    def _pred():
        pred_ref[:, :] = jax.lax.dot_general(
            h_ref[:, :], cls_ref[:, :], (((1,), (1,)), ((), ())),
            preferred_element_type=jnp.float32)

    @pl.when(i == _KSTEPS)
    def _finale():
        mrow = m_ref[:, :]
        srow = s_ref[:, :]
        big = jnp.max(mrow)
        w = srow * jnp.exp(mrow - big)
        total = jnp.sum(w) * (1.0 / 128.0)
        scales = jnp.exp(mrow - big) * (1.0 / total)
        for j in range(_NBLKS):
            sv = jnp.max(scales[:, j * 128:(j + 1) * 128])
            e2 = e_ref[pl.ds(j * _FROWS, _FROWS), :]
            seg = (e2 * sv).reshape(1, _FBLK)
            width = min(_FBLK, _N - j * _FBLK)
            attn_ref[:, pl.ds(j * _FBLK, width)] = seg[:, :width]


@jax.jit
def kernel(hidden_state, theta_classification, theta_rank):
    h = hidden_state.reshape(1, _D)
    cls_t = theta_classification.T
    pred, attn = pl.pallas_call(
        _tc_fused,
        grid=(_KSTEPS + 1,),
        in_specs=[
            pl.BlockSpec((1, _D), lambda i: (0, 0)),
            pl.BlockSpec((_T, _D), lambda i: (0, 0)),
        ] + [
            pl.BlockSpec(
                (_D, _FBLK),
                functools.partial(
                    lambda s, i: (0, s * _KSTEPS + jnp.minimum(i, _KSTEPS - 1)),
                    s))
            for s in range(_NSTREAMS)
        ],
        out_specs=[
            pl.BlockSpec((1, _T), lambda i: (0, 0)),
            pl.BlockSpec((1, _N), lambda i: (0, 0)),
        ],
        out_shape=[
            jax.ShapeDtypeStruct((1, _T), jnp.float32),
            jax.ShapeDtypeStruct((1, _N), jnp.float32),
        ],
        scratch_shapes=[
            pltpu.VMEM((_NBLKS * _FROWS, 128), jnp.float32),
            pltpu.VMEM((1, _NBLKS * 128), jnp.float32),
            pltpu.VMEM((1, _NBLKS * 128), jnp.float32),
        ],
    )(h, cls_t, theta_rank, theta_rank)
    return (pred, attn)
